# baseline (speedup 1.0000x reference)
import functools

import jax
import jax.numpy as jnp
import numpy as np
from jax import lax
from jax.experimental import pallas as pl
from jax.experimental.pallas import tpu as pltpu
from jax.experimental.pallas import tpu_sc as plsc

N = 10000
NN = 2 * N
D = 128
EQ = 320000
EC = 20000

NC, NS = 2, 16
CH = 128
INTRA_PT = 21504
CROSS_PT = 1536
INTRA_TOT = INTRA_PT * NS
CROSS_TOT = CROSS_PT * NS
ACC_N = 10096
ZROWS = ACC_N // NS
ICHUNKS = INTRA_PT // CH
XCHUNKS = CROSS_PT // CH

_mesh = plsc.VectorSubcoreMesh(core_axis_name="c", subcore_axis_name="s",
                               num_cores=NC, num_subcores=NS)


def _edge_pipeline(g, s, table, idxv, n_chunks, idx, isem, rows, gsem, ssem,
                   acc):
    def fire_idx(i, p):
        pltpu.async_copy(idxv.at[g, s * n_chunks + i], idx[p], isem[p])

    def wait_idx(p):
        pltpu.make_async_copy(idxv.at[g, 0], idx[p], isem[p]).wait()

    def fire_gather(p, r):
        pltpu.async_copy(table.at[idx[p].at[0]], rows[r], gsem[r])

    def wait_gather(p, r):
        pltpu.make_async_copy(table.at[idx[p].at[0]], rows[r],
                              gsem[r]).wait()

    def fire_scatter(p, r):
        pltpu.async_copy(rows[r], acc.at[idx[p].at[1]], ssem[r], add=True)

    def wait_scatter(p, r):
        pltpu.make_async_copy(rows[r], acc.at[idx[p].at[1]], ssem[r]).wait()

    fire_idx(0, 0)
    fire_idx(1, 1)
    wait_idx(0)
    fire_gather(0, 0)

    def body(k, carry):
        for b in range(12):
            i = 12 * k + b
            p, r = b % 4, b % 3
            wait_gather(p, r)

            @pl.when(i >= 2)
            def _():
                wait_scatter((b - 2) % 4, (b - 2) % 3)

            @pl.when(i + 2 < n_chunks)
            def _():
                fire_idx(i + 2, (b + 2) % 4)

            @pl.when(i + 1 < n_chunks)
            def _():
                wait_idx((b + 1) % 4)
                fire_gather((b + 1) % 4, (b + 1) % 3)

            fire_scatter(p, r)
        return carry

    lax.fori_loop(0, n_chunks // 12, body, 0)
    for j in (n_chunks - 2, n_chunks - 1):
        wait_scatter(j % 4, j % 3)


def _acc_zero(s, zeros, acc):
    pltpu.sync_copy(zeros, acc.at[pl.ds(s * ZROWS, ZROWS)])
    plsc.subcore_barrier()


def _acc_epilogue(g, s, acc, out):
    plsc.subcore_barrier()
    pltpu.sync_copy(acc.at[pl.ds(s * 624, 624)],
                    out.at[pl.ds(g * N + s * 624, 624)])

    @pl.when(s == NS - 1)
    def _():
        pltpu.sync_copy(acc.at[pl.ds(NS * 624, N - NS * 624)],
                        out.at[pl.ds(g * N + NS * 624, N - NS * 624)])


@functools.partial(
    pl.kernel,
    out_type=jax.ShapeDtypeStruct((NN, D), jnp.float32),
    mesh=_mesh,
    scratch_types=[
        [pltpu.VMEM((2, 128), jnp.int32)] * 4,
        [pltpu.SemaphoreType.DMA] * 4,
        [pltpu.VMEM((CH, D), jnp.float32)] * 3,
        [pltpu.SemaphoreType.DMA] * 3,
        [pltpu.SemaphoreType.DMA] * 3,
        pltpu.VMEM_SHARED((ACC_N, D), jnp.float32),
    ],
)
def _sc_phase2(scaled, unscaled, zeros, iidx, xidx, out,
               idx, isem, rows, gsem, ssem, acc):
    g = lax.axis_index("c")
    s = lax.axis_index("s")
    _acc_zero(s, zeros, acc)
    _edge_pipeline(g, s, scaled, iidx, ICHUNKS, idx, isem, rows, gsem, ssem,
                   acc)
    _edge_pipeline(g, s, unscaled, xidx, XCHUNKS, idx, isem, rows, gsem, ssem,
                   acc)
    _acc_epilogue(g, s, acc, out)


@functools.partial(
    pl.kernel,
    out_type=jax.ShapeDtypeStruct((NN, D), jnp.float32),
    mesh=_mesh,
    scratch_types=[
        [pltpu.VMEM((2, 128), jnp.int32)] * 4,
        [pltpu.SemaphoreType.DMA] * 4,
        [pltpu.VMEM((CH, D), jnp.float32)] * 3,
        [pltpu.SemaphoreType.DMA] * 3,
        [pltpu.SemaphoreType.DMA] * 3,
        pltpu.VMEM_SHARED((ACC_N, D), jnp.float32),
    ],
)
def _sc_phase1(unscaled, zeros, xidx, out, idx, isem, rows, gsem, ssem, acc):
    g = lax.axis_index("c")
    s = lax.axis_index("s")
    _acc_zero(s, zeros, acc)
    _edge_pipeline(g, s, unscaled, xidx, XCHUNKS, idx, isem, rows, gsem, ssem,
                   acc)
    _acc_epilogue(g, s, acc, out)


def _tc_layer(acc, mprev, W, b, msk, nrm, *, apply_elu, out_y, out_scaled,
              table_from_max):
    RB = 1000
    n_out = 1 + int(out_y) + int(out_scaled)

    def body(acc_ref, mp_ref, W_ref, b_ref, msk_ref, nrm_ref, *outs):
        y = jnp.dot(acc_ref[...], W_ref[...],
                    preferred_element_type=jnp.float32) + b_ref[...]
        if apply_elu:
            y = jnp.where(y > 0, y, jnp.exp(y) - 1.0)
        y = y * msk_ref[...]
        m = jnp.maximum(mp_ref[...], y)
        outs[0][...] = m
        k = 1
        if out_y:
            outs[k][...] = y
            k += 1
        if out_scaled:
            outs[k][...] = (m if table_from_max else y) * nrm_ref[...]

    blk = lambda r, c: pl.BlockSpec((r, c), lambda i: (i, 0))
    fixed = lambda r, c: pl.BlockSpec((r, c), lambda i: (0, 0))
    outs = pl.pallas_call(
        body,
        grid=(NN // RB,),
        in_specs=[blk(RB, D), blk(RB, D), fixed(D, D), fixed(1, D),
                  blk(RB, 1), blk(RB, 1)],
        out_specs=[blk(RB, D)] * n_out,
        out_shape=[jax.ShapeDtypeStruct((NN, D), jnp.float32)] * n_out,
    )(acc, mprev, W, b, msk, nrm)
    return outs


def _pack_idx(src, dst, per_tile, pad_src_base):
    tot = per_tile * NS
    npad = tot - src.shape[0]
    ps = jnp.asarray(pad_src_base + (np.arange(npad) % N), jnp.int32)
    pd = jnp.asarray(N + (np.arange(npad) % (ACC_N - N)), jnp.int32)
    s = jnp.concatenate([src, ps]).reshape(tot // CH, 1, 128)
    d = jnp.concatenate([dst, pd]).reshape(tot // CH, 1, 128)
    return jnp.concatenate([s, d], axis=1)


def kernel(Xq, edge_indexq, Xt, edge_indext, nn_map, cs_map, candidate_map,
           norm_q, norm_t, u2v_li, node_mask, cache_embeddings,
           W0, b0, W1, b1, W2, b2):
    Ws = [W0, W1, W2]
    bs = [b0[None, :], b1[None, :], b2[None, :]]

    eq = edge_indexq.astype(jnp.int32)
    et = edge_indext.astype(jnp.int32)
    uv = u2v_li.astype(jnp.int32)

    xidx = jnp.stack([_pack_idx(uv[1] + N, uv[0], CROSS_PT, N),
                      _pack_idx(uv[0], uv[1], CROSS_PT, 0)])

    iidx = jnp.stack([_pack_idx(eq[0], eq[1], INTRA_PT, 0),
                      _pack_idx(et[0] + N, et[1], INTRA_PT, N)])

    zeros = jnp.zeros((ZROWS, D), jnp.float32)
    msk = jnp.concatenate([jnp.ones((N,), jnp.float32),
                           1.0 - node_mask.astype(jnp.float32)])[:, None]
    nrm = jnp.concatenate([norm_q, norm_t])[:, None]

    m = jnp.concatenate([Xq, Xt], axis=0)

    for i in range(3):
        acc = _sc_phase1(m, zeros, xidx)
        res = _tc_layer(acc, m, Ws[i], bs[i], msk, nrm,
                        apply_elu=(i != 2), out_y=False, out_scaled=(i == 2),
                        table_from_max=True)
        m = res[0]
    scaled = res[1]

    y = m
    for i in range(3):
        acc = _sc_phase2(scaled, y, zeros, iidx, xidx)
        res = _tc_layer(acc, m, Ws[i], bs[i], msk, nrm,
                        apply_elu=(i != 2), out_y=(i != 2),
                        out_scaled=(i != 2), table_from_max=False)
        m = res[0]
        if i != 2:
            y, scaled = res[1], res[2]

    return (m[:N], m[N:])


# speedup vs baseline: 1.0377x; 1.0377x over previous
import functools

import jax
import jax.numpy as jnp
import numpy as np
from jax import lax
from jax.experimental import pallas as pl
from jax.experimental.pallas import tpu as pltpu
from jax.experimental.pallas import tpu_sc as plsc

N = 10000
NN = 2 * N
D = 128
EQ = 320000
EC = 20000

NC, NS = 2, 16
CH = 128
INTRA_PT = 20480
CROSS_PT = 1536
INTRA_TOT = INTRA_PT * NS
CROSS_TOT = CROSS_PT * NS
ACC_N = 10240
ZROWS = ACC_N // NS
ICHUNKS = INTRA_PT // CH
XCHUNKS = CROSS_PT // CH

_mesh = plsc.VectorSubcoreMesh(core_axis_name="c", subcore_axis_name="s",
                               num_cores=NC, num_subcores=NS)


def _edge_pipeline(g, s, table, idxv, n_chunks, idx, isem, rows, gsem, ssem,
                   acc):
    def fire_idx(i, p):
        pltpu.async_copy(idxv.at[g, s * n_chunks + i], idx[p], isem[p])

    def wait_idx(p):
        pltpu.make_async_copy(idxv.at[g, 0], idx[p], isem[p]).wait()

    def fire_gather(p, b):
        pltpu.async_copy(table.at[idx[p].at[0]], rows[b], gsem[b])

    def wait_gather(p, b):
        pltpu.make_async_copy(table.at[idx[p].at[0]], rows[b],
                              gsem[b]).wait()

    def fire_scatter(p, b):
        pltpu.async_copy(rows[b], acc.at[idx[p].at[1]], ssem[b], add=True)

    def wait_scatter(p, b):
        pltpu.make_async_copy(rows[b], acc.at[idx[p].at[1]], ssem[b]).wait()

    fire_idx(0, 0)
    fire_idx(1, 1)
    wait_idx(0)
    fire_gather(0, 0)

    def body(k, carry):
        for b in (0, 1, 2, 3):
            i = 4 * k + b
            p, b2 = b, b % 2
            wait_gather(p, b2)
            fire_scatter(p, b2)

            @pl.when(i >= 1)
            def _():
                wait_scatter((b - 1) % 4, 1 - b2)

            @pl.when(i + 2 < n_chunks)
            def _():
                fire_idx(i + 2, (b + 2) % 4)

            @pl.when(i + 1 < n_chunks)
            def _():
                wait_idx((b + 1) % 4)
                fire_gather((b + 1) % 4, 1 - b2)
        return carry

    lax.fori_loop(0, n_chunks // 4, body, 0)
    wait_scatter((n_chunks - 1) % 4, (n_chunks - 1) % 2)


def _acc_zero(s, zeros, acc):
    pltpu.sync_copy(zeros, acc.at[pl.ds(s * ZROWS, ZROWS)])
    plsc.subcore_barrier()


def _acc_epilogue(g, s, acc, out):
    plsc.subcore_barrier()
    pltpu.sync_copy(acc.at[pl.ds(s * 624, 624)],
                    out.at[pl.ds(g * N + s * 624, 624)])

    @pl.when(s == NS - 1)
    def _():
        pltpu.sync_copy(acc.at[pl.ds(NS * 624, N - NS * 624)],
                        out.at[pl.ds(g * N + NS * 624, N - NS * 624)])


@functools.partial(
    pl.kernel,
    out_type=jax.ShapeDtypeStruct((NN, D), jnp.float32),
    mesh=_mesh,
    scratch_types=[
        [pltpu.VMEM((2, 128), jnp.int32)] * 4,
        [pltpu.SemaphoreType.DMA] * 4,
        [pltpu.VMEM((CH, D), jnp.float32)] * 2,
        [pltpu.SemaphoreType.DMA] * 2,
        [pltpu.SemaphoreType.DMA] * 2,
        pltpu.VMEM_SHARED((ACC_N, D), jnp.float32),
    ],
)
def _sc_phase2(scaled, unscaled, zeros, iidx, xidx, out,
               idx, isem, rows, gsem, ssem, acc):
    g = lax.axis_index("c")
    s = lax.axis_index("s")
    _acc_zero(s, zeros, acc)
    _edge_pipeline(g, s, scaled, iidx, ICHUNKS, idx, isem, rows, gsem, ssem,
                   acc)
    _edge_pipeline(g, s, unscaled, xidx, XCHUNKS, idx, isem, rows, gsem, ssem,
                   acc)
    _acc_epilogue(g, s, acc, out)


@functools.partial(
    pl.kernel,
    out_type=jax.ShapeDtypeStruct((NN, D), jnp.float32),
    mesh=_mesh,
    scratch_types=[
        [pltpu.VMEM((2, 128), jnp.int32)] * 4,
        [pltpu.SemaphoreType.DMA] * 4,
        [pltpu.VMEM((CH, D), jnp.float32)] * 2,
        [pltpu.SemaphoreType.DMA] * 2,
        [pltpu.SemaphoreType.DMA] * 2,
        pltpu.VMEM_SHARED((ACC_N, D), jnp.float32),
    ],
)
def _sc_phase1(unscaled, zeros, xidx, out, idx, isem, rows, gsem, ssem, acc):
    g = lax.axis_index("c")
    s = lax.axis_index("s")
    _acc_zero(s, zeros, acc)
    _edge_pipeline(g, s, unscaled, xidx, XCHUNKS, idx, isem, rows, gsem, ssem,
                   acc)
    _acc_epilogue(g, s, acc, out)


def _tc_layer(acc, mprev, W, b, msk, nrm, *, apply_elu, out_y, out_scaled,
              table_from_max):
    RB = 1000
    n_out = 1 + int(out_y) + int(out_scaled)

    def body(acc_ref, mp_ref, W_ref, b_ref, msk_ref, nrm_ref, *outs):
        y = jnp.dot(acc_ref[...], W_ref[...],
                    preferred_element_type=jnp.float32) + b_ref[...]
        if apply_elu:
            y = jnp.where(y > 0, y, jnp.exp(y) - 1.0)
        y = y * msk_ref[...]
        m = jnp.maximum(mp_ref[...], y)
        outs[0][...] = m
        k = 1
        if out_y:
            outs[k][...] = y
            k += 1
        if out_scaled:
            outs[k][...] = (m if table_from_max else y) * nrm_ref[...]

    blk = lambda r, c: pl.BlockSpec((r, c), lambda i: (i, 0))
    fixed = lambda r, c: pl.BlockSpec((r, c), lambda i: (0, 0))
    outs = pl.pallas_call(
        body,
        grid=(NN // RB,),
        in_specs=[blk(RB, D), blk(RB, D), fixed(D, D), fixed(1, D),
                  blk(RB, 1), blk(RB, 1)],
        out_specs=[blk(RB, D)] * n_out,
        out_shape=[jax.ShapeDtypeStruct((NN, D), jnp.float32)] * n_out,
    )(acc, mprev, W, b, msk, nrm)
    return outs


def _pack_idx(src, dst, per_tile, pad_src_base):
    tot = per_tile * NS
    npad = tot - src.shape[0]
    ps = jnp.asarray(pad_src_base + (np.arange(npad) % N), jnp.int32)
    pd = jnp.asarray(N + (np.arange(npad) % (ACC_N - N)), jnp.int32)
    s = jnp.concatenate([src, ps]).reshape(tot // CH, 1, 128)
    d = jnp.concatenate([dst, pd]).reshape(tot // CH, 1, 128)
    return jnp.concatenate([s, d], axis=1)


def kernel(Xq, edge_indexq, Xt, edge_indext, nn_map, cs_map, candidate_map,
           norm_q, norm_t, u2v_li, node_mask, cache_embeddings,
           W0, b0, W1, b1, W2, b2):
    Ws = [W0, W1, W2]
    bs = [b0[None, :], b1[None, :], b2[None, :]]

    eq = edge_indexq.astype(jnp.int32)
    et = edge_indext.astype(jnp.int32)
    uv = u2v_li.astype(jnp.int32)

    xidx = jnp.stack([_pack_idx(uv[1] + N, uv[0], CROSS_PT, N),
                      _pack_idx(uv[0], uv[1], CROSS_PT, 0)])

    iidx = jnp.stack([_pack_idx(eq[0], eq[1], INTRA_PT, 0),
                      _pack_idx(et[0] + N, et[1], INTRA_PT, N)])

    zeros = jnp.zeros((ZROWS, D), jnp.float32)
    msk = jnp.concatenate([jnp.ones((N,), jnp.float32),
                           1.0 - node_mask.astype(jnp.float32)])[:, None]
    nrm = jnp.concatenate([norm_q, norm_t])[:, None]

    m = jnp.concatenate([Xq, Xt], axis=0)

    for i in range(3):
        acc = _sc_phase1(m, zeros, xidx)
        res = _tc_layer(acc, m, Ws[i], bs[i], msk, nrm,
                        apply_elu=(i != 2), out_y=False, out_scaled=(i == 2),
                        table_from_max=True)
        m = res[0]
    scaled = res[1]

    y = m
    for i in range(3):
        acc = _sc_phase2(scaled, y, zeros, iidx, xidx)
        res = _tc_layer(acc, m, Ws[i], bs[i], msk, nrm,
                        apply_elu=(i != 2), out_y=(i != 2),
                        out_scaled=(i != 2), table_from_max=False)
        m = res[0]
        if i != 2:
            y, scaled = res[1], res[2]

    return (m[:N], m[N:])


# speedup vs baseline: 1.0620x; 1.0235x over previous
import functools

import jax
import jax.numpy as jnp
import numpy as np
from jax import lax
from jax.experimental import pallas as pl
from jax.experimental.pallas import tpu as pltpu
from jax.experimental.pallas import tpu_sc as plsc

N = 10000
NN = 2 * N
D = 128
EQ = 320000
EC = 20000

NC, NS = 2, 16
CH = 128
INTRA_PT = 20480
CROSS_PT = 1536
INTRA_TOT = INTRA_PT * NS
CROSS_TOT = CROSS_PT * NS
ACC_N = 10240
ZROWS = ACC_N // NS
ICHUNKS = INTRA_PT // CH
XCHUNKS = CROSS_PT // CH

_mesh = plsc.VectorSubcoreMesh(core_axis_name="c", subcore_axis_name="s",
                               num_cores=NC, num_subcores=NS)


def _edge_pipeline(g, s, table, idxv, n_chunks, idx, isem, rows, gsem, ssem,
                   acc):
    def fire_idx(i, p):
        pltpu.async_copy(idxv.at[g, s * n_chunks + i], idx[p], isem[p])

    def wait_idx(p):
        pltpu.make_async_copy(idxv.at[g, 0], idx[p], isem[p]).wait()

    def fire_gather(p, b):
        pltpu.async_copy(table.at[idx[p].at[0]], rows[b], gsem[b])

    def wait_gather(p, b):
        pltpu.make_async_copy(table.at[idx[p].at[0]], rows[b],
                              gsem[b]).wait()

    def fire_scatter(p, b):
        pltpu.async_copy(rows[b], acc.at[idx[p].at[1]], ssem[b], add=True)

    def wait_scatter(p, b):
        pltpu.make_async_copy(rows[b], acc.at[idx[p].at[1]], ssem[b]).wait()

    fire_idx(0, 0)
    fire_idx(1, 1)
    wait_idx(0)
    fire_gather(0, 0)

    def body(k, carry):
        for b in (0, 1, 2, 3):
            i = 4 * k + b
            p, b2 = b, b % 2
            wait_gather(p, b2)
            fire_scatter(p, b2)

            @pl.when(i >= 1)
            def _():
                wait_scatter((b - 1) % 4, 1 - b2)

            @pl.when(i + 2 < n_chunks)
            def _():
                fire_idx(i + 2, (b + 2) % 4)

            @pl.when(i + 1 < n_chunks)
            def _():
                wait_idx((b + 1) % 4)
                fire_gather((b + 1) % 4, 1 - b2)
        return carry

    lax.fori_loop(0, n_chunks // 4, body, 0)
    wait_scatter((n_chunks - 1) % 4, (n_chunks - 1) % 2)


def _acc_zero(s, zeros, acc):
    pltpu.sync_copy(zeros, acc.at[pl.ds(s * ZROWS, ZROWS)])
    plsc.subcore_barrier()


def _acc_epilogue(g, s, acc, out):
    plsc.subcore_barrier()
    pltpu.sync_copy(acc.at[pl.ds(s * 624, 624)],
                    out.at[pl.ds(g * N + s * 624, 624)])

    @pl.when(s == NS - 1)
    def _():
        pltpu.sync_copy(acc.at[pl.ds(NS * 624, N - NS * 624)],
                        out.at[pl.ds(g * N + NS * 624, N - NS * 624)])


@functools.partial(
    pl.kernel,
    out_type=jax.ShapeDtypeStruct((NN, D), jnp.float32),
    mesh=_mesh,
    scratch_types=[
        [pltpu.VMEM((2, 128), jnp.int32)] * 4,
        [pltpu.SemaphoreType.DMA] * 4,
        [pltpu.VMEM((CH, D), jnp.float32)] * 2,
        [pltpu.SemaphoreType.DMA] * 2,
        [pltpu.SemaphoreType.DMA] * 2,
        pltpu.VMEM_SHARED((ACC_N, D), jnp.float32),
    ],
)
def _sc_phase2(scaled, unscaled, zeros, iidx, xidx, out,
               idx, isem, rows, gsem, ssem, acc):
    g = lax.axis_index("c")
    s = lax.axis_index("s")
    _acc_zero(s, zeros, acc)
    _edge_pipeline(g, s, scaled, iidx, ICHUNKS, idx, isem, rows, gsem, ssem,
                   acc)
    _edge_pipeline(g, s, unscaled, xidx, XCHUNKS, idx, isem, rows, gsem, ssem,
                   acc)
    _acc_epilogue(g, s, acc, out)


@functools.partial(
    pl.kernel,
    out_type=jax.ShapeDtypeStruct((NN, D), jnp.float32),
    mesh=_mesh,
    scratch_types=[
        [pltpu.VMEM((2, 128), jnp.int32)] * 4,
        [pltpu.SemaphoreType.DMA] * 4,
        [pltpu.VMEM((CH, D), jnp.float32)] * 2,
        [pltpu.SemaphoreType.DMA] * 2,
        [pltpu.SemaphoreType.DMA] * 2,
        pltpu.VMEM_SHARED((ACC_N, D), jnp.float32),
    ],
)
def _sc_phase1(unscaled, zeros, xidx, out, idx, isem, rows, gsem, ssem, acc):
    g = lax.axis_index("c")
    s = lax.axis_index("s")
    _acc_zero(s, zeros, acc)
    _edge_pipeline(g, s, unscaled, xidx, XCHUNKS, idx, isem, rows, gsem, ssem,
                   acc)
    _acc_epilogue(g, s, acc, out)


def _tc_layer(acc, mprev, W, b, msk, nrm, *, apply_elu, out_y, out_scaled,
              table_from_max):
    RB = 2000
    n_out = 1 + int(out_y) + int(out_scaled)

    def body(acc_ref, mp_ref, W_ref, b_ref, msk_ref, nrm_ref, *outs):
        y = jnp.dot(acc_ref[...], W_ref[...],
                    preferred_element_type=jnp.float32) + b_ref[...]
        if apply_elu:
            y = jnp.where(y > 0, y, jnp.exp(y) - 1.0)
        y = y * msk_ref[...]
        m = jnp.maximum(mp_ref[...], y)
        outs[0][...] = m
        k = 1
        if out_y:
            outs[k][...] = y
            k += 1
        if out_scaled:
            outs[k][...] = (m if table_from_max else y) * nrm_ref[...]

    blk = lambda r, c: pl.BlockSpec((r, c), lambda i: (i, 0))
    fixed = lambda r, c: pl.BlockSpec((r, c), lambda i: (0, 0))
    outs = pl.pallas_call(
        body,
        grid=(NN // RB,),
        in_specs=[blk(RB, D), blk(RB, D), fixed(D, D), fixed(1, D),
                  blk(RB, 1), blk(RB, 1)],
        out_specs=[blk(RB, D)] * n_out,
        out_shape=[jax.ShapeDtypeStruct((NN, D), jnp.float32)] * n_out,
    )(acc, mprev, W, b, msk, nrm)
    return outs


def _pack_idx(src, dst, per_tile, pad_src_base):
    tot = per_tile * NS
    npad = tot - src.shape[0]
    ps = jnp.asarray(pad_src_base + (np.arange(npad) % N), jnp.int32)
    pd = jnp.asarray(N + (np.arange(npad) % (ACC_N - N)), jnp.int32)
    s = jnp.concatenate([src, ps]).reshape(tot // CH, 1, 128)
    d = jnp.concatenate([dst, pd]).reshape(tot // CH, 1, 128)
    return jnp.concatenate([s, d], axis=1)


def kernel(Xq, edge_indexq, Xt, edge_indext, nn_map, cs_map, candidate_map,
           norm_q, norm_t, u2v_li, node_mask, cache_embeddings,
           W0, b0, W1, b1, W2, b2):
    Ws = [W0, W1, W2]
    bs = [b0[None, :], b1[None, :], b2[None, :]]

    eq = edge_indexq.astype(jnp.int32)
    et = edge_indext.astype(jnp.int32)
    uv = u2v_li.astype(jnp.int32)

    xidx = jnp.stack([_pack_idx(uv[1] + N, uv[0], CROSS_PT, N),
                      _pack_idx(uv[0], uv[1], CROSS_PT, 0)])

    iidx = jnp.stack([_pack_idx(eq[0], eq[1], INTRA_PT, 0),
                      _pack_idx(et[0] + N, et[1], INTRA_PT, N)])

    zeros = jnp.zeros((ZROWS, D), jnp.float32)
    msk = jnp.concatenate([jnp.ones((N,), jnp.float32),
                           1.0 - node_mask.astype(jnp.float32)])[:, None]
    nrm = jnp.concatenate([norm_q, norm_t])[:, None]

    m = jnp.concatenate([Xq, Xt], axis=0)

    for i in range(3):
        acc = _sc_phase1(m, zeros, xidx)
        res = _tc_layer(acc, m, Ws[i], bs[i], msk, nrm,
                        apply_elu=(i != 2), out_y=False, out_scaled=(i == 2),
                        table_from_max=True)
        m = res[0]
    scaled = res[1]

    y = m
    for i in range(3):
        acc = _sc_phase2(scaled, y, zeros, iidx, xidx)
        res = _tc_layer(acc, m, Ws[i], bs[i], msk, nrm,
                        apply_elu=(i != 2), out_y=(i != 2),
                        out_scaled=(i != 2), table_from_max=False)
        m = res[0]
        if i != 2:
            y, scaled = res[1], res[2]

    return (m[:N], m[N:])


# speedup vs baseline: 1.0694x; 1.0069x over previous
import functools

import jax
import jax.numpy as jnp
import numpy as np
from jax import lax
from jax.experimental import pallas as pl
from jax.experimental.pallas import tpu as pltpu
from jax.experimental.pallas import tpu_sc as plsc

N = 10000
NN = 2 * N
D = 128
EQ = 320000
EC = 20000

NC, NS = 2, 16
CH = 128
INTRA_PT = 20480
CROSS_PT = 1536
INTRA_TOT = INTRA_PT * NS
CROSS_TOT = CROSS_PT * NS
ACC_N = 10240
ZROWS = ACC_N // NS
ICHUNKS = INTRA_PT // CH
XCHUNKS = CROSS_PT // CH

_mesh = plsc.VectorSubcoreMesh(core_axis_name="c", subcore_axis_name="s",
                               num_cores=NC, num_subcores=NS)


def _edge_pipeline(g, s, table, idxv, n_chunks, idx, isem, rows, gsem, ssem,
                   acc):
    def fire_idx(i, p):
        pltpu.async_copy(idxv.at[g, s * n_chunks + i], idx[p], isem[p])

    def wait_idx(p):
        pltpu.make_async_copy(idxv.at[g, 0], idx[p], isem[p]).wait()

    def fire_gather(p, b):
        pltpu.async_copy(table.at[idx[p].at[0]], rows[b], gsem[b])

    def wait_gather(p, b):
        pltpu.make_async_copy(table.at[idx[p].at[0]], rows[b],
                              gsem[b]).wait()

    def fire_scatter(p, b):
        pltpu.async_copy(rows[b], acc.at[idx[p].at[1]], ssem[b], add=True)

    def wait_scatter(p, b):
        pltpu.make_async_copy(rows[b], acc.at[idx[p].at[1]], ssem[b]).wait()

    fire_idx(0, 0)
    fire_idx(1, 1)
    wait_idx(0)
    fire_gather(0, 0)

    def body(k, carry):
        for b in (0, 1, 2, 3):
            i = 4 * k + b
            p, b2 = b, b % 2
            wait_gather(p, b2)
            fire_scatter(p, b2)

            @pl.when(i >= 1)
            def _():
                wait_scatter((b - 1) % 4, 1 - b2)

            @pl.when(i + 2 < n_chunks)
            def _():
                fire_idx(i + 2, (b + 2) % 4)

            @pl.when(i + 1 < n_chunks)
            def _():
                wait_idx((b + 1) % 4)
                fire_gather((b + 1) % 4, 1 - b2)
        return carry

    lax.fori_loop(0, n_chunks // 4, body, 0)
    wait_scatter((n_chunks - 1) % 4, (n_chunks - 1) % 2)


def _acc_zero(s, zeros, acc):
    pltpu.sync_copy(zeros, acc.at[pl.ds(s * ZROWS, ZROWS)])
    plsc.subcore_barrier()


def _acc_epilogue(g, s, acc, out):
    plsc.subcore_barrier()
    pltpu.sync_copy(acc.at[pl.ds(s * 624, 624)],
                    out.at[pl.ds(g * N + s * 624, 624)])

    @pl.when(s == NS - 1)
    def _():
        pltpu.sync_copy(acc.at[pl.ds(NS * 624, N - NS * 624)],
                        out.at[pl.ds(g * N + NS * 624, N - NS * 624)])


@functools.partial(
    pl.kernel,
    out_type=jax.ShapeDtypeStruct((NN, D), jnp.float32),
    mesh=_mesh,
    scratch_types=[
        [pltpu.VMEM((2, 128), jnp.int32)] * 4,
        [pltpu.SemaphoreType.DMA] * 4,
        [pltpu.VMEM((CH, D), jnp.float32)] * 2,
        [pltpu.SemaphoreType.DMA] * 2,
        [pltpu.SemaphoreType.DMA] * 2,
        pltpu.VMEM_SHARED((ACC_N, D), jnp.float32),
    ],
)
def _sc_phase2(scaled, unscaled, zeros, iidx, xidx, out,
               idx, isem, rows, gsem, ssem, acc):
    g = lax.axis_index("c")
    s = lax.axis_index("s")
    _acc_zero(s, zeros, acc)
    _edge_pipeline(g, s, scaled, iidx, ICHUNKS, idx, isem, rows, gsem, ssem,
                   acc)
    _edge_pipeline(g, s, unscaled, xidx, XCHUNKS, idx, isem, rows, gsem, ssem,
                   acc)
    _acc_epilogue(g, s, acc, out)


@functools.partial(
    pl.kernel,
    out_type=jax.ShapeDtypeStruct((NN, D), jnp.float32),
    mesh=_mesh,
    scratch_types=[
        [pltpu.VMEM((2, 128), jnp.int32)] * 4,
        [pltpu.SemaphoreType.DMA] * 4,
        [pltpu.VMEM((CH, D), jnp.float32)] * 2,
        [pltpu.SemaphoreType.DMA] * 2,
        [pltpu.SemaphoreType.DMA] * 2,
        pltpu.VMEM_SHARED((ACC_N, D), jnp.float32),
    ],
)
def _sc_phase1(unscaled, zeros, xidx, out, idx, isem, rows, gsem, ssem, acc):
    g = lax.axis_index("c")
    s = lax.axis_index("s")
    _acc_zero(s, zeros, acc)
    _edge_pipeline(g, s, unscaled, xidx, XCHUNKS, idx, isem, rows, gsem, ssem,
                   acc)
    _acc_epilogue(g, s, acc, out)


def _tc_layer(acc, mprev, W, b, msk, nrm, *, apply_elu, out_y, out_scaled,
              table_from_max):
    RB = 4000
    n_out = 1 + int(out_y) + int(out_scaled)

    def body(acc_ref, mp_ref, W_ref, b_ref, msk_ref, nrm_ref, *outs):
        y = jnp.dot(acc_ref[...], W_ref[...],
                    preferred_element_type=jnp.float32) + b_ref[...]
        if apply_elu:
            y = jnp.where(y > 0, y, jnp.exp(y) - 1.0)
        y = y * msk_ref[...]
        m = jnp.maximum(mp_ref[...], y)
        outs[0][...] = m
        k = 1
        if out_y:
            outs[k][...] = y
            k += 1
        if out_scaled:
            outs[k][...] = (m if table_from_max else y) * nrm_ref[...]

    blk = lambda r, c: pl.BlockSpec((r, c), lambda i: (i, 0))
    fixed = lambda r, c: pl.BlockSpec((r, c), lambda i: (0, 0))
    outs = pl.pallas_call(
        body,
        grid=(NN // RB,),
        in_specs=[blk(RB, D), blk(RB, D), fixed(D, D), fixed(1, D),
                  blk(RB, 1), blk(RB, 1)],
        out_specs=[blk(RB, D)] * n_out,
        out_shape=[jax.ShapeDtypeStruct((NN, D), jnp.float32)] * n_out,
    )(acc, mprev, W, b, msk, nrm)
    return outs


def _pack_idx(src, dst, per_tile, pad_src_base):
    tot = per_tile * NS
    npad = tot - src.shape[0]
    ps = jnp.asarray(pad_src_base + (np.arange(npad) % N), jnp.int32)
    pd = jnp.asarray(N + (np.arange(npad) % (ACC_N - N)), jnp.int32)
    s = jnp.concatenate([src, ps]).reshape(tot // CH, 1, 128)
    d = jnp.concatenate([dst, pd]).reshape(tot // CH, 1, 128)
    return jnp.concatenate([s, d], axis=1)


def kernel(Xq, edge_indexq, Xt, edge_indext, nn_map, cs_map, candidate_map,
           norm_q, norm_t, u2v_li, node_mask, cache_embeddings,
           W0, b0, W1, b1, W2, b2):
    Ws = [W0, W1, W2]
    bs = [b0[None, :], b1[None, :], b2[None, :]]

    eq = edge_indexq.astype(jnp.int32)
    et = edge_indext.astype(jnp.int32)
    uv = u2v_li.astype(jnp.int32)

    xidx = jnp.stack([_pack_idx(uv[1] + N, uv[0], CROSS_PT, N),
                      _pack_idx(uv[0], uv[1], CROSS_PT, 0)])

    iidx = jnp.stack([_pack_idx(eq[0], eq[1], INTRA_PT, 0),
                      _pack_idx(et[0] + N, et[1], INTRA_PT, N)])

    zeros = jnp.zeros((ZROWS, D), jnp.float32)
    msk = jnp.concatenate([jnp.ones((N,), jnp.float32),
                           1.0 - node_mask.astype(jnp.float32)])[:, None]
    nrm = jnp.concatenate([norm_q, norm_t])[:, None]

    m = jnp.concatenate([Xq, Xt], axis=0)

    for i in range(3):
        acc = _sc_phase1(m, zeros, xidx)
        res = _tc_layer(acc, m, Ws[i], bs[i], msk, nrm,
                        apply_elu=(i != 2), out_y=False, out_scaled=(i == 2),
                        table_from_max=True)
        m = res[0]
    scaled = res[1]

    y = m
    for i in range(3):
        acc = _sc_phase2(scaled, y, zeros, iidx, xidx)
        res = _tc_layer(acc, m, Ws[i], bs[i], msk, nrm,
                        apply_elu=(i != 2), out_y=(i != 2),
                        out_scaled=(i != 2), table_from_max=False)
        m = res[0]
        if i != 2:
            y, scaled = res[1], res[2]

    return (m[:N], m[N:])


# speedup vs baseline: 1.0752x; 1.0055x over previous
import functools

import jax
import jax.numpy as jnp
import numpy as np
from jax import lax
from jax.experimental import pallas as pl
from jax.experimental.pallas import tpu as pltpu
from jax.experimental.pallas import tpu_sc as plsc

N = 10000
NN = 2 * N
D = 128
EQ = 320000
EC = 20000

NC, NS = 2, 16
CH = 128
INTRA_PT = 20480
CROSS_PT = 1536
INTRA_TOT = INTRA_PT * NS
CROSS_TOT = CROSS_PT * NS
ACC_N = 10240
ZROWS = ACC_N // NS
ICHUNKS = INTRA_PT // CH
XCHUNKS = CROSS_PT // CH

_mesh = plsc.VectorSubcoreMesh(core_axis_name="c", subcore_axis_name="s",
                               num_cores=NC, num_subcores=NS)


def _pipeline_prime(g, s, table, idxv, n_chunks, idx, isem, rows, gsem):
    pltpu.async_copy(idxv.at[g, s * n_chunks + 0], idx[0], isem[0])
    pltpu.async_copy(idxv.at[g, s * n_chunks + 1], idx[1], isem[1])
    pltpu.make_async_copy(idxv.at[g, 0], idx[0], isem[0]).wait()
    pltpu.async_copy(table.at[idx[0].at[0]], rows[0], gsem[0])


def _edge_pipeline(g, s, table, idxv, n_chunks, idx, isem, rows, gsem, ssem,
                   acc, primed=False):
    def fire_idx(i, p):
        pltpu.async_copy(idxv.at[g, s * n_chunks + i], idx[p], isem[p])

    def wait_idx(p):
        pltpu.make_async_copy(idxv.at[g, 0], idx[p], isem[p]).wait()

    def fire_gather(p, b):
        pltpu.async_copy(table.at[idx[p].at[0]], rows[b], gsem[b])

    def wait_gather(p, b):
        pltpu.make_async_copy(table.at[idx[p].at[0]], rows[b],
                              gsem[b]).wait()

    def fire_scatter(p, b):
        pltpu.async_copy(rows[b], acc.at[idx[p].at[1]], ssem[b], add=True)

    def wait_scatter(p, b):
        pltpu.make_async_copy(rows[b], acc.at[idx[p].at[1]], ssem[b]).wait()

    if not primed:
        fire_idx(0, 0)
        fire_idx(1, 1)
        wait_idx(0)
        fire_gather(0, 0)

    def body(k, carry):
        for b in (0, 1, 2, 3):
            i = 4 * k + b
            p, b2 = b, b % 2
            wait_gather(p, b2)
            fire_scatter(p, b2)

            @pl.when(i >= 1)
            def _():
                wait_scatter((b - 1) % 4, 1 - b2)

            @pl.when(i + 2 < n_chunks)
            def _():
                fire_idx(i + 2, (b + 2) % 4)

            @pl.when(i + 1 < n_chunks)
            def _():
                wait_idx((b + 1) % 4)
                fire_gather((b + 1) % 4, 1 - b2)
        return carry

    lax.fori_loop(0, n_chunks // 4, body, 0)
    wait_scatter((n_chunks - 1) % 4, (n_chunks - 1) % 2)


def _acc_zero(s, zeros, acc):
    pltpu.sync_copy(zeros, acc.at[pl.ds(s * ZROWS, ZROWS)])
    plsc.subcore_barrier()


def _acc_epilogue(g, s, acc, out):
    plsc.subcore_barrier()
    pltpu.sync_copy(acc.at[pl.ds(s * 624, 624)],
                    out.at[pl.ds(g * N + s * 624, 624)])

    @pl.when(s == NS - 1)
    def _():
        pltpu.sync_copy(acc.at[pl.ds(NS * 624, N - NS * 624)],
                        out.at[pl.ds(g * N + NS * 624, N - NS * 624)])


@functools.partial(
    pl.kernel,
    out_type=jax.ShapeDtypeStruct((NN, D), jnp.float32),
    mesh=_mesh,
    scratch_types=[
        [pltpu.VMEM((2, 128), jnp.int32)] * 4,
        [pltpu.SemaphoreType.DMA] * 4,
        [pltpu.VMEM((CH, D), jnp.float32)] * 2,
        [pltpu.SemaphoreType.DMA] * 2,
        [pltpu.SemaphoreType.DMA] * 2,
        pltpu.VMEM_SHARED((ACC_N, D), jnp.float32),
    ],
)
def _sc_phase2(scaled, unscaled, zeros, iidx, xidx, out,
               idx, isem, rows, gsem, ssem, acc):
    g = lax.axis_index("c")
    s = lax.axis_index("s")
    _pipeline_prime(g, s, scaled, iidx, ICHUNKS, idx, isem, rows, gsem)
    _acc_zero(s, zeros, acc)
    _edge_pipeline(g, s, scaled, iidx, ICHUNKS, idx, isem, rows, gsem, ssem,
                   acc, primed=True)
    _edge_pipeline(g, s, unscaled, xidx, XCHUNKS, idx, isem, rows, gsem, ssem,
                   acc)
    _acc_epilogue(g, s, acc, out)


@functools.partial(
    pl.kernel,
    out_type=jax.ShapeDtypeStruct((NN, D), jnp.float32),
    mesh=_mesh,
    scratch_types=[
        [pltpu.VMEM((2, 128), jnp.int32)] * 4,
        [pltpu.SemaphoreType.DMA] * 4,
        [pltpu.VMEM((CH, D), jnp.float32)] * 2,
        [pltpu.SemaphoreType.DMA] * 2,
        [pltpu.SemaphoreType.DMA] * 2,
        pltpu.VMEM_SHARED((ACC_N, D), jnp.float32),
    ],
)
def _sc_phase1(unscaled, zeros, xidx, out, idx, isem, rows, gsem, ssem, acc):
    g = lax.axis_index("c")
    s = lax.axis_index("s")
    _pipeline_prime(g, s, unscaled, xidx, XCHUNKS, idx, isem, rows, gsem)
    _acc_zero(s, zeros, acc)
    _edge_pipeline(g, s, unscaled, xidx, XCHUNKS, idx, isem, rows, gsem, ssem,
                   acc, primed=True)
    _acc_epilogue(g, s, acc, out)


def _tc_layer(acc, mprev, W, b, msk, nrm, *, apply_elu, out_y, out_scaled,
              table_from_max):
    RB = 4000
    n_out = 1 + int(out_y) + int(out_scaled)

    def body(acc_ref, mp_ref, W_ref, b_ref, msk_ref, nrm_ref, *outs):
        y = jnp.dot(acc_ref[...], W_ref[...],
                    preferred_element_type=jnp.float32) + b_ref[...]
        if apply_elu:
            y = jnp.where(y > 0, y, jnp.exp(y) - 1.0)
        y = y * msk_ref[...]
        m = jnp.maximum(mp_ref[...], y)
        outs[0][...] = m
        k = 1
        if out_y:
            outs[k][...] = y
            k += 1
        if out_scaled:
            outs[k][...] = (m if table_from_max else y) * nrm_ref[...]

    blk = lambda r, c: pl.BlockSpec((r, c), lambda i: (i, 0))
    fixed = lambda r, c: pl.BlockSpec((r, c), lambda i: (0, 0))
    outs = pl.pallas_call(
        body,
        grid=(NN // RB,),
        in_specs=[blk(RB, D), blk(RB, D), fixed(D, D), fixed(1, D),
                  blk(RB, 1), blk(RB, 1)],
        out_specs=[blk(RB, D)] * n_out,
        out_shape=[jax.ShapeDtypeStruct((NN, D), jnp.float32)] * n_out,
    )(acc, mprev, W, b, msk, nrm)
    return outs


def _pack_idx(src, dst, per_tile, pad_src_base):
    tot = per_tile * NS
    npad = tot - src.shape[0]
    ps = jnp.asarray(pad_src_base + (np.arange(npad) % N), jnp.int32)
    pd = jnp.asarray(N + (np.arange(npad) % (ACC_N - N)), jnp.int32)
    s = jnp.concatenate([src, ps]).reshape(tot // CH, 1, 128)
    d = jnp.concatenate([dst, pd]).reshape(tot // CH, 1, 128)
    return jnp.concatenate([s, d], axis=1)


def kernel(Xq, edge_indexq, Xt, edge_indext, nn_map, cs_map, candidate_map,
           norm_q, norm_t, u2v_li, node_mask, cache_embeddings,
           W0, b0, W1, b1, W2, b2):
    Ws = [W0, W1, W2]
    bs = [b0[None, :], b1[None, :], b2[None, :]]

    eq = edge_indexq.astype(jnp.int32)
    et = edge_indext.astype(jnp.int32)
    uv = u2v_li.astype(jnp.int32)

    xidx = jnp.stack([_pack_idx(uv[1] + N, uv[0], CROSS_PT, N),
                      _pack_idx(uv[0], uv[1], CROSS_PT, 0)])

    iidx = jnp.stack([_pack_idx(eq[0], eq[1], INTRA_PT, 0),
                      _pack_idx(et[0] + N, et[1], INTRA_PT, N)])

    zeros = jnp.zeros((ZROWS, D), jnp.float32)
    msk = jnp.concatenate([jnp.ones((N,), jnp.float32),
                           1.0 - node_mask.astype(jnp.float32)])[:, None]
    nrm = jnp.concatenate([norm_q, norm_t])[:, None]

    m = jnp.concatenate([Xq, Xt], axis=0)

    for i in range(3):
        acc = _sc_phase1(m, zeros, xidx)
        res = _tc_layer(acc, m, Ws[i], bs[i], msk, nrm,
                        apply_elu=(i != 2), out_y=False, out_scaled=(i == 2),
                        table_from_max=True)
        m = res[0]
    scaled = res[1]

    y = m
    for i in range(3):
        acc = _sc_phase2(scaled, y, zeros, iidx, xidx)
        res = _tc_layer(acc, m, Ws[i], bs[i], msk, nrm,
                        apply_elu=(i != 2), out_y=(i != 2),
                        out_scaled=(i != 2), table_from_max=False)
        m = res[0]
        if i != 2:
            y, scaled = res[1], res[2]

    return (m[:N], m[N:])
